# baseline (speedup 1.0000x reference)
import functools

import jax
import jax.numpy as jnp
from jax import lax
from jax.experimental import pallas as pl
from jax.experimental.pallas import tpu as pltpu
from jax.experimental.pallas import tpu_sc as plsc

N_NODES = 10000
N_EDGES = 160000
NUM_TILES = 16
NUM_CORES = 2
EDGE_BLOCK = 128
EDGES_PER_TILE = 10240
N_HALVES = 2
NB_H = EDGES_PER_TILE // (N_HALVES * EDGE_BLOCK)
E_PAD = NUM_TILES * EDGES_PER_TILE
ACC_ROWS = 10112
INIT_ROWS = ACC_ROWS // NUM_TILES
OUT_ROWS = 624
DUMMY_ROW = N_NODES


def _sc_agg_body(n_pairs, h_ref, srcp_ref, dst_ref, init_ref, out_ref,
                 acc, src_v, dst_v, rows_a, rows_b,
                 sem_a, sem_b, sem_sa, sem_sb):
    cid = lax.axis_index("c")
    tid = lax.axis_index("s")

    for p in range(n_pairs):
        pltpu.sync_copy(init_ref, acc.at[pl.ds(tid * INIT_ROWS, INIT_ROWS)])
        plsc.subcore_barrier()

        for half in range(N_HALVES):
            pltpu.sync_copy(srcp_ref.at[p, cid, tid, half], src_v)
            pltpu.sync_copy(dst_ref.at[tid, half], dst_v)
            pltpu.async_copy(h_ref.at[src_v.at[0]], rows_a, sem_a)
            pltpu.async_copy(h_ref.at[src_v.at[1]], rows_b, sem_b)

            def body(i, carry):
                j = 2 * i
                pltpu.make_async_copy(
                    h_ref.at[src_v.at[j]], rows_a, sem_a).wait()
                pltpu.async_copy(
                    rows_a, acc.at[dst_v.at[j]], sem_sa, add=True)
                pltpu.make_async_copy(
                    h_ref.at[src_v.at[j + 1]], rows_b, sem_b).wait()
                pltpu.async_copy(
                    rows_b, acc.at[dst_v.at[j + 1]], sem_sb, add=True)
                pltpu.make_async_copy(
                    rows_a, acc.at[dst_v.at[j]], sem_sa).wait()
                pltpu.async_copy(h_ref.at[src_v.at[j + 2]], rows_a, sem_a)
                pltpu.make_async_copy(
                    rows_b, acc.at[dst_v.at[j + 1]], sem_sb).wait()
                pltpu.async_copy(h_ref.at[src_v.at[j + 3]], rows_b, sem_b)
                return carry
            lax.fori_loop(0, (NB_H - 2) // 2, body, 0)
            j = NB_H - 2
            pltpu.make_async_copy(h_ref.at[src_v.at[j]], rows_a, sem_a).wait()
            pltpu.sync_copy(rows_a, acc.at[dst_v.at[j]], add=True)
            pltpu.make_async_copy(
                h_ref.at[src_v.at[j + 1]], rows_b, sem_b).wait()
            pltpu.sync_copy(rows_b, acc.at[dst_v.at[j + 1]], add=True)

        plsc.subcore_barrier()
        obase = (2 * p) * N_NODES + cid * N_NODES

        @pl.when(tid < NUM_TILES - 1)
        def _():
            pltpu.sync_copy(
                acc.at[pl.ds(tid * OUT_ROWS, OUT_ROWS)],
                out_ref.at[pl.ds(obase + tid * OUT_ROWS, OUT_ROWS)])

        @pl.when(tid == NUM_TILES - 1)
        def _():
            last = (NUM_TILES - 1) * OUT_ROWS
            pltpu.sync_copy(
                acc.at[pl.ds(last, N_NODES - last)],
                out_ref.at[pl.ds(obase + last, N_NODES - last)])

        if p + 1 < n_pairs:
            plsc.subcore_barrier()


def _make_sc_aggregate(n_pairs):
    return functools.partial(
        pl.kernel,
        out_type=jax.ShapeDtypeStruct(
            (n_pairs * NUM_CORES * N_NODES, 128), jnp.float32),
        mesh=plsc.VectorSubcoreMesh(core_axis_name="c", subcore_axis_name="s"),
        scratch_types=[
            pltpu.VMEM_SHARED((ACC_ROWS, 128), jnp.float32),
            pltpu.VMEM((NB_H, EDGE_BLOCK), jnp.int32),
            pltpu.VMEM((NB_H, EDGE_BLOCK), jnp.int32),
            pltpu.VMEM((EDGE_BLOCK, 128), jnp.float32),
            pltpu.VMEM((EDGE_BLOCK, 128), jnp.float32),
            pltpu.SemaphoreType.DMA,
            pltpu.SemaphoreType.DMA,
            pltpu.SemaphoreType.DMA,
            pltpu.SemaphoreType.DMA,
        ],
    )(functools.partial(_sc_agg_body, n_pairs))


_sc_aggregate_1 = _make_sc_aggregate(1)
_sc_aggregate_4 = _make_sc_aggregate(4)


def _mm_body(oc, bn, relu, a_ref, w_ref, b_ref, o_ref):
    c = pl.program_id(1)
    nc = pl.num_programs(1)

    @pl.when(c == 0)
    def _():
        for o in range(oc):
            o_ref[o] = jnp.broadcast_to(b_ref[o], (bn, 128))

    m = jnp.dot(a_ref[...], w_ref[...], preferred_element_type=jnp.float32)
    for o in range(oc):
        o_ref[o] += m[:, o * 128:(o + 1) * 128]

    if relu:
        @pl.when(c == nc - 1)
        def _():
            for o in range(oc):
                o_ref[o] = jnp.maximum(o_ref[o], 0.0)


def _tc_matmul(a3, w3, bias, relu):
    cc, n, _ = a3.shape
    o_full = w3.shape[2]
    oc = o_full // 128
    bn = 1000
    grid = (n // bn, cc)
    bias3 = bias.reshape(oc, 1, 128)
    return pl.pallas_call(
        functools.partial(_mm_body, oc, bn, relu),
        grid=grid,
        in_specs=[
            pl.BlockSpec((None, bn, 128), lambda nb, c: (c, nb, 0)),
            pl.BlockSpec((None, 128, o_full), lambda nb, c: (c, 0, 0)),
            pl.BlockSpec((oc, 1, 128), lambda nb, c: (0, 0, 0)),
        ],
        out_specs=pl.BlockSpec((oc, bn, 128), lambda nb, c: (0, nb, 0)),
        out_shape=jax.ShapeDtypeStruct((oc, n, 128), jnp.float32),
    )(a3, w3, bias3)


def kernel(features, edge_index, W0, b0, W1, b1, W2, b2):
    n, f_in = features.shape
    src = edge_index[0]
    dst = edge_index[1]
    pad = E_PAD - N_EDGES
    src_p = jnp.concatenate([src, jnp.zeros((pad,), jnp.int32)])
    dst_p = jnp.concatenate([dst, jnp.full((pad,), DUMMY_ROW, jnp.int32)])
    chunk_off = (jnp.arange(4)[:, None] * 2 + jnp.arange(2)[None, :]) * n
    srcp_all = (src_p[None, None, :] + chunk_off[:, :, None]).reshape(
        4, NUM_CORES, NUM_TILES, N_HALVES, NB_H, EDGE_BLOCK)
    srcp_1 = srcp_all[:1]
    dst3 = dst_p.reshape(NUM_TILES, N_HALVES, NB_H, EDGE_BLOCK)
    zero_init = jnp.zeros((INIT_ROWS, 128), jnp.float32)
    b2_init = jnp.broadcast_to(b2, (INIT_ROWS, 128))

    x3 = features.reshape(n, 2, 128).transpose(1, 0, 2)
    a0 = _sc_aggregate_1(x3.reshape(2 * n, 128), srcp_1, dst3, zero_init)
    a0 = a0.reshape(2, n, 128)
    h1 = _tc_matmul(a0, W0.reshape(2, 128, -1), b0, relu=True)

    a1 = _sc_aggregate_4(h1.reshape(8 * n, 128), srcp_all, dst3, zero_init)
    a1 = a1.reshape(8, n, 128)
    h2 = _tc_matmul(a1, W1.reshape(8, 128, -1), b1, relu=True)

    t = _tc_matmul(h2, W2.reshape(8, 128, -1), jnp.zeros((128,), jnp.float32),
                   relu=False)
    tcat = jnp.concatenate([t[0], t[0]], axis=0)
    out = _sc_aggregate_1(tcat, srcp_1, dst3, b2_init)
    return out[:n]


# speedup vs baseline: 1.0159x; 1.0159x over previous
import functools

import jax
import jax.numpy as jnp
from jax import lax
from jax.experimental import pallas as pl
from jax.experimental.pallas import tpu as pltpu
from jax.experimental.pallas import tpu_sc as plsc

N_NODES = 10000
N_EDGES = 160000
NUM_TILES = 16
NUM_CORES = 2
EDGE_BLOCK = 64
EDGES_PER_TILE = 10240
N_STAGES = 4
NB_Q = EDGES_PER_TILE // (N_STAGES * EDGE_BLOCK)
NBUF = 4
E_PAD = NUM_TILES * EDGES_PER_TILE
ACC_ROWS = 10112
INIT_ROWS = ACC_ROWS // NUM_TILES
OUT_ROWS = 624
DUMMY_ROW = N_NODES


def _sc_agg_body(n_pairs, h_ref, srcp_ref, dst_ref, init_ref, out_ref,
                 acc, src_v, dst_v, r0, r1, r2, r3,
                 g0, g1, g2, g3, s0, s1, s2, s3):
    cid = lax.axis_index("c")
    tid = lax.axis_index("s")
    rows = (r0, r1, r2, r3)
    sem_g = (g0, g1, g2, g3)
    sem_s = (s0, s1, s2, s3)

    for p in range(n_pairs):
        pltpu.sync_copy(init_ref, acc.at[pl.ds(tid * INIT_ROWS, INIT_ROWS)])
        plsc.subcore_barrier()

        for q in range(N_STAGES):
            pltpu.sync_copy(srcp_ref.at[p, cid, tid, q], src_v)
            pltpu.sync_copy(dst_ref.at[tid, q], dst_v)
            for b in range(NBUF):
                pltpu.async_copy(h_ref.at[src_v.at[b]], rows[b], sem_g[b])

            def body(g, carry):
                for b in range(NBUF):
                    j = NBUF * g + b
                    pltpu.make_async_copy(
                        h_ref.at[src_v.at[j]], rows[b], sem_g[b]).wait()
                    pltpu.async_copy(
                        rows[b], acc.at[dst_v.at[j]], sem_s[b], add=True)
                for b in range(NBUF):
                    j = NBUF * g + b
                    pltpu.make_async_copy(
                        rows[b], acc.at[dst_v.at[j]], sem_s[b]).wait()
                    pltpu.async_copy(
                        h_ref.at[src_v.at[j + NBUF]], rows[b], sem_g[b])
                return carry
            lax.fori_loop(0, NB_Q // NBUF - 1, body, 0)
            jd = NB_Q - NBUF
            for b in range(NBUF):
                pltpu.make_async_copy(
                    h_ref.at[src_v.at[jd + b]], rows[b], sem_g[b]).wait()
                pltpu.async_copy(
                    rows[b], acc.at[dst_v.at[jd + b]], sem_s[b], add=True)
            for b in range(NBUF):
                pltpu.make_async_copy(
                    rows[b], acc.at[dst_v.at[jd + b]], sem_s[b]).wait()

        plsc.subcore_barrier()
        obase = (2 * p) * N_NODES + cid * N_NODES

        @pl.when(tid < NUM_TILES - 1)
        def _():
            pltpu.sync_copy(
                acc.at[pl.ds(tid * OUT_ROWS, OUT_ROWS)],
                out_ref.at[pl.ds(obase + tid * OUT_ROWS, OUT_ROWS)])

        @pl.when(tid == NUM_TILES - 1)
        def _():
            last = (NUM_TILES - 1) * OUT_ROWS
            pltpu.sync_copy(
                acc.at[pl.ds(last, N_NODES - last)],
                out_ref.at[pl.ds(obase + last, N_NODES - last)])

        if p + 1 < n_pairs:
            plsc.subcore_barrier()


def _make_sc_aggregate(n_pairs):
    return functools.partial(
        pl.kernel,
        out_type=jax.ShapeDtypeStruct(
            (n_pairs * NUM_CORES * N_NODES, 128), jnp.float32),
        mesh=plsc.VectorSubcoreMesh(core_axis_name="c", subcore_axis_name="s"),
        scratch_types=(
            [pltpu.VMEM_SHARED((ACC_ROWS, 128), jnp.float32),
             pltpu.VMEM((NB_Q, EDGE_BLOCK), jnp.int32),
             pltpu.VMEM((NB_Q, EDGE_BLOCK), jnp.int32)]
            + [pltpu.VMEM((EDGE_BLOCK, 128), jnp.float32)] * NBUF
            + [pltpu.SemaphoreType.DMA] * (2 * NBUF)
        ),
    )(functools.partial(_sc_agg_body, n_pairs))


_sc_aggregate_1 = _make_sc_aggregate(1)
_sc_aggregate_4 = _make_sc_aggregate(4)


def _mm_body(oc, bn, relu, a_ref, w_ref, b_ref, o_ref):
    c = pl.program_id(1)
    nc = pl.num_programs(1)

    @pl.when(c == 0)
    def _():
        for o in range(oc):
            o_ref[o] = jnp.broadcast_to(b_ref[o], (bn, 128))

    m = jnp.dot(a_ref[...], w_ref[...], preferred_element_type=jnp.float32)
    for o in range(oc):
        o_ref[o] += m[:, o * 128:(o + 1) * 128]

    if relu:
        @pl.when(c == nc - 1)
        def _():
            for o in range(oc):
                o_ref[o] = jnp.maximum(o_ref[o], 0.0)


def _tc_matmul(a3, w3, bias, relu):
    cc, n, _ = a3.shape
    o_full = w3.shape[2]
    oc = o_full // 128
    bn = 1000
    grid = (n // bn, cc)
    bias3 = bias.reshape(oc, 1, 128)
    return pl.pallas_call(
        functools.partial(_mm_body, oc, bn, relu),
        grid=grid,
        in_specs=[
            pl.BlockSpec((None, bn, 128), lambda nb, c: (c, nb, 0)),
            pl.BlockSpec((None, 128, o_full), lambda nb, c: (c, 0, 0)),
            pl.BlockSpec((oc, 1, 128), lambda nb, c: (0, 0, 0)),
        ],
        out_specs=pl.BlockSpec((oc, bn, 128), lambda nb, c: (0, nb, 0)),
        out_shape=jax.ShapeDtypeStruct((oc, n, 128), jnp.float32),
    )(a3, w3, bias3)


def kernel(features, edge_index, W0, b0, W1, b1, W2, b2):
    n, f_in = features.shape
    src = edge_index[0]
    dst = edge_index[1]
    pad = E_PAD - N_EDGES
    src_p = jnp.concatenate([src, jnp.zeros((pad,), jnp.int32)])
    dst_p = jnp.concatenate([dst, jnp.full((pad,), DUMMY_ROW, jnp.int32)])
    chunk_off = (jnp.arange(4)[:, None] * 2 + jnp.arange(2)[None, :]) * n
    srcp_all = (src_p[None, None, :] + chunk_off[:, :, None]).reshape(
        4, NUM_CORES, NUM_TILES, N_STAGES, NB_Q, EDGE_BLOCK)
    srcp_1 = srcp_all[:1]
    dst3 = dst_p.reshape(NUM_TILES, N_STAGES, NB_Q, EDGE_BLOCK)
    zero_init = jnp.zeros((INIT_ROWS, 128), jnp.float32)
    b2_init = jnp.broadcast_to(b2, (INIT_ROWS, 128))

    x3 = features.reshape(n, 2, 128).transpose(1, 0, 2)
    a0 = _sc_aggregate_1(x3.reshape(2 * n, 128), srcp_1, dst3, zero_init)
    a0 = a0.reshape(2, n, 128)
    h1 = _tc_matmul(a0, W0.reshape(2, 128, -1), b0, relu=True)

    a1 = _sc_aggregate_4(h1.reshape(8 * n, 128), srcp_all, dst3, zero_init)
    a1 = a1.reshape(8, n, 128)
    h2 = _tc_matmul(a1, W1.reshape(8, 128, -1), b1, relu=True)

    t = _tc_matmul(h2, W2.reshape(8, 128, -1), jnp.zeros((128,), jnp.float32),
                   relu=False)
    tcat = jnp.concatenate([t[0], t[0]], axis=0)
    out = _sc_aggregate_1(tcat, srcp_1, dst3, b2_init)
    return out[:n]
